# Initial kernel scaffold; baseline (speedup 1.0000x reference)
import jax
import jax.numpy as jnp
from jax.experimental import pallas as pl


def kernel(Y, A, W):
    raise NotImplementedError("write your pallas kernel here")


# speedup vs baseline: 4.3452x; 4.3452x over previous
import jax
import jax.numpy as jnp
from jax import lax
from jax.experimental import pallas as pl
from jax.experimental.pallas import tpu as pltpu

_S_TOPK = 64
_TAU = 0.1
_ETA = 0.5
_N_SOFT_LAYERS = 3


def _body(y_ref, a_ref, w_ref, o_ref):
    batch, m = y_ref.shape
    _, n = a_ref.shape
    f32 = jnp.float32

    a = a_ref[...]
    gram = lax.dot_general(a, a, (((0,), (0,)), ((), ())),
                           preferred_element_type=f32)
    ii = lax.broadcasted_iota(jnp.int32, (n, n), 0)
    jj = lax.broadcasted_iota(jnp.int32, (n, n), 1)
    mm = jnp.where(ii == jj, f32(1.0), f32(0.0)) - f32(_ETA) * gram

    y = y_ref[...]
    bt = f32(_ETA) * lax.dot_general(y, a, (((1,), (0,)), ((), ())),
                                     preferred_element_type=f32)
    w = w_ref[...]

    lane = lax.broadcasted_iota(jnp.int32, (batch, n), 1)
    inv_tau = f32(1.0 / _TAU)
    big = jnp.int32(2 ** 30)

    xt = jnp.zeros((batch, n), dtype=f32)
    for layer in range(_N_SOFT_LAYERS):
        if layer == 0:
            ht = bt
        else:
            ht = bt + lax.dot_general(xt, mm, (((1,), (0,)), ((), ())),
                                      preferred_element_type=f32)
        s0 = jnp.abs(ht * w)

        def step(_, carry):
            s_work, mask = carry
            mx = jnp.max(s_work, axis=1, keepdims=True)
            row = jnp.exp(-jnp.abs(s0 - mx) * inv_tau)
            z = jnp.sum(row, axis=1, keepdims=True)
            mask = mask + row / z
            ismax = s_work == mx
            first = jnp.min(jnp.where(ismax, lane, big), axis=1, keepdims=True)
            s_work = jnp.where(lane == first, f32(-jnp.inf), s_work)
            return s_work, mask

        _, mask = lax.fori_loop(
            0, _S_TOPK, step, (s0, jnp.zeros((batch, n), dtype=f32)))
        xt = mask * ht

    o_ref[...] = xt


def kernel(Y, A, W):
    batch, _ = Y.shape
    _, n = A.shape
    return pl.pallas_call(
        _body,
        out_shape=jax.ShapeDtypeStruct((batch, n), jnp.float32),
    )(Y, A, W.reshape(1, n))


# speedup vs baseline: 5.8574x; 1.3480x over previous
import jax
import jax.numpy as jnp
from jax import lax
from jax.experimental import pallas as pl
from jax.experimental.pallas import tpu as pltpu

_S_TOPK = 64
_TAU = 0.1
_ETA = 0.5
_N_SOFT_LAYERS = 3
_L = 128


def _cmpx(x, d, keep_max):
    size = x.shape[-1]
    lower = (lax.broadcasted_iota(jnp.int32, x.shape, x.ndim - 1) & d) == 0
    partner = jnp.where(lower,
                        pltpu.roll(x, size - d, x.ndim - 1),
                        pltpu.roll(x, d, x.ndim - 1))
    return jnp.where(keep_max, jnp.maximum(x, partner),
                     jnp.minimum(x, partner))


def _asc_flag(shape):
    c = shape[1]
    if c == 1:
        return jnp.zeros(shape, dtype=jnp.bool_)
    sub = lax.broadcasted_iota(jnp.int32, shape, 1)
    return sub >= (c // 2)


def _sort128(x):
    asc = _asc_flag(x.shape)
    i = lax.broadcasted_iota(jnp.int32, x.shape, x.ndim - 1)
    for p in range(1, 8):
        block_desc = ((i >> p) & 1) == 0
        for q in range(p - 1, -1, -1):
            d = 1 << q
            keep_max = (((i & d) == 0) == block_desc) != asc
            x = _cmpx(x, d, keep_max)
    return x


def _clean128(x):
    asc = _asc_flag(x.shape)
    i = lax.broadcasted_iota(jnp.int32, x.shape, x.ndim - 1)
    for q in range(6, -1, -1):
        d = 1 << q
        keep_max = ((i & d) == 0) != asc
        x = _cmpx(x, d, keep_max)
    return x


def _top128(s3):
    x = _sort128(s3)
    while x.shape[1] > 1:
        h = x.shape[1] // 2
        x = _clean128(jnp.maximum(x[:, :h], x[:, h:]))
    return x


def _body(y_ref, a_ref, w_ref, o_ref):
    batch, m = y_ref.shape
    _, n = a_ref.shape
    f32 = jnp.float32

    a = a_ref[...]
    gram = lax.dot_general(a, a, (((0,), (0,)), ((), ())),
                           preferred_element_type=f32)
    ii = lax.broadcasted_iota(jnp.int32, (n, n), 0)
    jj = lax.broadcasted_iota(jnp.int32, (n, n), 1)
    mm = jnp.where(ii == jj, f32(1.0), f32(0.0)) - f32(_ETA) * gram

    y = y_ref[...]
    bt = f32(_ETA) * lax.dot_general(y, a, (((1,), (0,)), ((), ())),
                                     preferred_element_type=f32)
    w = w_ref[...]
    inv_tau = f32(1.0 / _TAU)

    xt = jnp.zeros((batch, n), dtype=f32)
    for layer in range(_N_SOFT_LAYERS):
        if layer == 0:
            ht = bt
        else:
            ht = bt + lax.dot_general(xt, mm, (((1,), (0,)), ((), ())),
                                      preferred_element_type=f32)
        s0 = jnp.abs(ht * w)

        t = _top128(s0.reshape(batch, n // _L, _L))
        t = t.reshape(batch, _L)

        mask = jnp.zeros((batch, n), dtype=f32)
        for i in range(_S_TOPK):
            ti = t[:, i:i + 1]
            e = jnp.exp(-jnp.abs(s0 - ti) * inv_tau)
            z = jnp.sum(e, axis=1, keepdims=True)
            mask = mask + e * (f32(1.0) / z)
        xt = mask * ht

    o_ref[...] = xt


def kernel(Y, A, W):
    batch, _ = Y.shape
    _, n = A.shape
    return pl.pallas_call(
        _body,
        out_shape=jax.ShapeDtypeStruct((batch, n), jnp.float32),
    )(Y, A, W.reshape(1, n))
